# baseline (speedup 1.0000x reference)
import functools

import jax
import jax.numpy as jnp
from jax import lax
from jax.experimental import pallas as pl
from jax.experimental.pallas import tpu as pltpu
from jax.experimental.pallas import tpu_sc as plsc

NC = 2
NS = 16
NW = NC * NS
LANES = 16
K = 16
NPC = 8


def _build_agg(n_pad, d, nch_a, nch_b, nbuf=2):
    dw = d // 2
    ipc = NPC * K
    nch_s = nch_a + nch_b
    cnt_a = -(-(nch_a + 8) // 8) * 8
    cnt_b = -(-(nch_b + 8) // 8) * 8
    assert nch_s * NS * NPC == n_pad
    assert nch_a % nbuf == 0 and nch_b % nbuf == 0 and nch_b >= nbuf
    inv_k = 1.0 / K

    mesh = plsc.VectorSubcoreMesh(
        core_axis_name="c", subcore_axis_name="s",
        num_cores=NC, num_subcores=NS)

    def body(x_hbm, adjs_hbm, out_hbm, idx_v, *scr):
        c = lax.axis_index("c")
        s = lax.axis_index("s")
        base_ch = s * nch_s + c * nch_a
        nch = nch_a + c * (nch_b - nch_a)
        base_nd = base_ch * NPC
        rows = scr[0:nbuf]
        stg = scr[nbuf:2 * nbuf]
        gsem = scr[2 * nbuf:3 * nbuf]
        osem = scr[3 * nbuf:4 * nbuf]

        base_al = (base_ch // 8) * 8
        off = base_ch - base_al

        @pl.when(c == 0)
        def _():
            pltpu.sync_copy(adjs_hbm.at[pl.ds(base_al, cnt_a)],
                            idx_v.at[pl.ds(0, cnt_a)])

        @pl.when(c == 1)
        def _():
            pltpu.sync_copy(adjs_hbm.at[pl.ds(base_al, cnt_b)],
                            idx_v.at[pl.ds(0, cnt_b)])

        def gather(j, b):
            return pltpu.make_async_copy(
                x_hbm.at[idx_v.at[off + j]], rows[b], gsem[b])

        def store(j, b):
            return pltpu.make_async_copy(
                stg[b], out_hbm.at[pl.ds(base_nd + j * NPC, NPC)], osem[b])

        for b in range(nbuf):
            gather(b, b).start()

        def outer(jg, _):
            for b in range(nbuf):
                j = jg * nbuf + b
                gather(j, b).wait()

                @pl.when(jg > 0)
                def _():
                    store(j - nbuf, b).wait()

                def slice_body(si, _):
                    colw = pl.ds(si * LANES, LANES)
                    for n in range(NPC):
                        acc_a, acc_b = plsc.unpack(
                            plsc.bitcast(rows[b][n * K, colw], jnp.bfloat16),
                            format=plsc.PackFormat.INTERLEAVED)
                        for k in range(1, K):
                            ua, ub = plsc.unpack(
                                plsc.bitcast(rows[b][n * K + k, colw],
                                             jnp.bfloat16),
                                format=plsc.PackFormat.INTERLEAVED)
                            acc_a = acc_a + ua
                            acc_b = acc_b + ub
                        stg[b][n, colw] = plsc.bitcast(
                            plsc.pack(acc_a * inv_k, acc_b * inv_k,
                                      format=plsc.PackFormat.INTERLEAVED),
                            jnp.int32)
                    return 0

                lax.fori_loop(0, dw // LANES, slice_body, 0, unroll=False)

                @pl.when(j + nbuf < nch)
                def _():
                    gather(j + nbuf, b).start()

                store(j, b).start()
            return 0

        lax.fori_loop(0, nch // nbuf, outer, 0, unroll=False)
        for b in range(nbuf):
            store(nch - nbuf + b, b).wait()

    return pl.kernel(
        body,
        out_type=jax.ShapeDtypeStruct((n_pad, dw), jnp.int32),
        mesh=mesh,
        compiler_params=pltpu.CompilerParams(needs_layout_passes=False),
        scratch_types=(
            [pltpu.VMEM((max(cnt_a, cnt_b), ipc), jnp.int32)]
            + [pltpu.VMEM((ipc, dw), jnp.int32) for _ in range(nbuf)]
            + [pltpu.VMEM((NPC, dw), jnp.int32) for _ in range(nbuf)]
            + [pltpu.SemaphoreType.DMA for _ in range(2 * nbuf)]
        ),
    )


def _unpack_pairs(w):
    lo = lax.bitcast_convert_type((w & 0xFFFF).astype(jnp.uint16),
                                  jnp.bfloat16)
    hi = lax.bitcast_convert_type(
        lax.shift_right_logical(w, 16).astype(jnp.uint16), jnp.bfloat16)
    return lo, hi


def _pack_pairs(ha, hb):
    ta = lax.bitcast_convert_type(
        ha.astype(jnp.bfloat16), jnp.uint16).astype(jnp.int32)
    tb = lax.bitcast_convert_type(
        hb.astype(jnp.bfloat16), jnp.uint16).astype(jnp.int32)
    return ta | lax.shift_left(tb, 16)


def _neigh_dots(acc, a_ref, wna_ref, wnb_ref):
    lo, hi = _unpack_pairs(a_ref[...])
    acc = acc + jnp.dot(lo, wna_ref[...], preferred_element_type=jnp.float32)
    acc = acc + jnp.dot(hi, wnb_ref[...], preferred_element_type=jnp.float32)
    return acc


def _mm1_kernel(x_ref, a_ref, ws_ref, wna_ref, wnb_ref, otab_ref):
    acc = jnp.dot(x_ref[...].astype(jnp.bfloat16), ws_ref[...],
                  preferred_element_type=jnp.float32)
    h = jnp.maximum(_neigh_dots(acc, a_ref, wna_ref, wnb_ref), 0.0)
    hdim = h.shape[1]
    otab_ref[...] = _pack_pairs(h[:, :hdim // 2], h[:, hdim // 2:])


@functools.partial(jax.jit, static_argnames=("bm",))
def _mm1(x, agg, ws, wna, wnb, bm=512):
    m_grid = agg.shape[0]
    d = x.shape[1]
    dw = agg.shape[1]
    h = ws.shape[1]
    return pl.pallas_call(
        _mm1_kernel,
        grid=(m_grid // bm,),
        in_specs=[
            pl.BlockSpec((bm, d), lambda i: (i, 0)),
            pl.BlockSpec((bm, dw), lambda i: (i, 0)),
            pl.BlockSpec((d, h), lambda i: (0, 0)),
            pl.BlockSpec((dw, h), lambda i: (0, 0)),
            pl.BlockSpec((dw, h), lambda i: (0, 0)),
        ],
        out_specs=pl.BlockSpec((bm, h // 2), lambda i: (i, 0)),
        out_shape=jax.ShapeDtypeStruct((m_grid, h // 2), jnp.int32),
    )(x, agg, ws, wna, wnb)


def _mm2_kernel(xt_ref, a_ref, wsa_ref, wsb_ref, wna_ref, wnb_ref, o_ref):
    lo_s, hi_s = _unpack_pairs(xt_ref[...])
    acc = jnp.dot(lo_s, wsa_ref[...], preferred_element_type=jnp.float32)
    acc = acc + jnp.dot(hi_s, wsb_ref[...],
                        preferred_element_type=jnp.float32)
    o_ref[...] = jnp.maximum(_neigh_dots(acc, a_ref, wna_ref, wnb_ref), 0.0)


@functools.partial(jax.jit, static_argnames=("bm", "m_out"))
def _mm2(xtab, agg, wsa, wsb, wna, wnb, bm=512, m_out=None):
    m_grid = agg.shape[0]
    dw = agg.shape[1]
    h = wsa.shape[1]
    m_out = m_grid if m_out is None else m_out
    return pl.pallas_call(
        _mm2_kernel,
        grid=(m_grid // bm,),
        in_specs=[
            pl.BlockSpec((bm, dw), lambda i: (i, 0)),
            pl.BlockSpec((bm, dw), lambda i: (i, 0)),
            pl.BlockSpec((dw, h), lambda i: (0, 0)),
            pl.BlockSpec((dw, h), lambda i: (0, 0)),
            pl.BlockSpec((dw, h), lambda i: (0, 0)),
            pl.BlockSpec((dw, h), lambda i: (0, 0)),
        ],
        out_specs=pl.BlockSpec((bm, h), lambda i: (i, 0)),
        out_shape=jax.ShapeDtypeStruct((m_out, h), jnp.float32),
    )(xtab, agg, wsa, wsb, wna, wnb)


SPLIT1 = (62, 18)
SPLIT2 = (58, 22)


def kernel(nodes, adjs, features, W1, W2):
    n, _ = adjs.shape
    d_in = features.shape[1]
    h1_dim = W1.shape[1]

    quantum = NW * NPC * 2
    n_pad = ((n + quantum - 1) // quantum) * quantum

    adjs_r = jnp.pad(adjs, ((0, n_pad - n), (0, 0))).reshape(-1, NPC * K)
    adjs_r = jnp.pad(adjs_r, ((0, 16), (0, 0)))

    dwi = d_in // 2
    tab1 = _pack_pairs(features[:, :dwi], features[:, dwi:])

    W1b = W1.astype(jnp.bfloat16)
    W2b = W2.astype(jnp.bfloat16)

    agg1 = _build_agg(n_pad, d_in, *SPLIT1)(tab1, adjs_r)
    tab2 = _mm1(features, agg1, W1b[:d_in], W1b[d_in:d_in + dwi],
                W1b[d_in + dwi:])

    agg2 = _build_agg(n_pad, h1_dim, *SPLIT2)(tab2, adjs_r)
    dwh = h1_dim // 2
    return _mm2(tab2, agg2, W2b[:dwh], W2b[dwh:h1_dim],
                W2b[h1_dim:h1_dim + dwh], W2b[h1_dim + dwh:], m_out=n)


# speedup vs baseline: 1.0062x; 1.0062x over previous
import functools

import jax
import jax.numpy as jnp
from jax import lax
from jax.experimental import pallas as pl
from jax.experimental.pallas import tpu as pltpu
from jax.experimental.pallas import tpu_sc as plsc

NC = 2
NS = 16
NW = NC * NS
LANES = 16
K = 16
NPC = 8


def _build_agg(n_pad, d, nch_a, nch_b, nbuf=2):
    dw = d // 2
    ipc = NPC * K
    nch_s = nch_a + nch_b
    cnt_a = -(-(nch_a + 8) // 8) * 8
    cnt_b = -(-(nch_b + 8) // 8) * 8
    assert nch_s * NS * NPC == n_pad
    assert nch_a % nbuf == 0 and nch_b % nbuf == 0 and nch_b >= nbuf
    inv_k = 1.0 / K

    mesh = plsc.VectorSubcoreMesh(
        core_axis_name="c", subcore_axis_name="s",
        num_cores=NC, num_subcores=NS)

    def body(x_hbm, adjs_hbm, out_hbm, idx_v, *scr):
        c = lax.axis_index("c")
        s = lax.axis_index("s")
        base_ch = s * nch_s + c * nch_a
        nch = nch_a + c * (nch_b - nch_a)
        base_nd = base_ch * NPC
        rows = scr[0:nbuf]
        stg = scr[nbuf:2 * nbuf]
        gsem = scr[2 * nbuf:3 * nbuf]
        osem = scr[3 * nbuf:4 * nbuf]

        base_al = (base_ch // 8) * 8
        off = base_ch - base_al

        @pl.when(c == 0)
        def _():
            pltpu.sync_copy(adjs_hbm.at[pl.ds(base_al, cnt_a)],
                            idx_v.at[pl.ds(0, cnt_a)])

        @pl.when(c == 1)
        def _():
            pltpu.sync_copy(adjs_hbm.at[pl.ds(base_al, cnt_b)],
                            idx_v.at[pl.ds(0, cnt_b)])

        def gather(j, b):
            return pltpu.make_async_copy(
                x_hbm.at[idx_v.at[off + j]], rows[b], gsem[b])

        def store(j, b):
            return pltpu.make_async_copy(
                stg[b], out_hbm.at[pl.ds(base_nd + j * NPC, NPC)], osem[b])

        for b in range(nbuf):
            gather(b, b).start()

        def outer(jg, _):
            for b in range(nbuf):
                j = jg * nbuf + b
                gather(j, b).wait()

                @pl.when(jg > 0)
                def _():
                    store(j - nbuf, b).wait()

                def slice_body(si, _):
                    colw = pl.ds(si * LANES, LANES)
                    for n in range(NPC):
                        acc_a, acc_b = plsc.unpack(
                            plsc.bitcast(rows[b][n * K, colw], jnp.bfloat16),
                            format=plsc.PackFormat.INTERLEAVED)
                        for k in range(1, K):
                            ua, ub = plsc.unpack(
                                plsc.bitcast(rows[b][n * K + k, colw],
                                             jnp.bfloat16),
                                format=plsc.PackFormat.INTERLEAVED)
                            acc_a = acc_a + ua
                            acc_b = acc_b + ub
                        stg[b][n, colw] = plsc.bitcast(
                            plsc.pack(acc_a * inv_k, acc_b * inv_k,
                                      format=plsc.PackFormat.INTERLEAVED),
                            jnp.int32)
                    return 0

                lax.fori_loop(0, dw // LANES, slice_body, 0, unroll=False)

                @pl.when(j + nbuf < nch)
                def _():
                    gather(j + nbuf, b).start()

                store(j, b).start()
            return 0

        lax.fori_loop(0, nch // nbuf, outer, 0, unroll=False)
        for b in range(nbuf):
            store(nch - nbuf + b, b).wait()

    return pl.kernel(
        body,
        out_type=jax.ShapeDtypeStruct((n_pad, dw), jnp.int32),
        mesh=mesh,
        compiler_params=pltpu.CompilerParams(needs_layout_passes=False),
        scratch_types=(
            [pltpu.VMEM((max(cnt_a, cnt_b), ipc), jnp.int32)]
            + [pltpu.VMEM((ipc, dw), jnp.int32) for _ in range(nbuf)]
            + [pltpu.VMEM((NPC, dw), jnp.int32) for _ in range(nbuf)]
            + [pltpu.SemaphoreType.DMA for _ in range(2 * nbuf)]
        ),
    )


def _unpack_pairs(w):
    lo = lax.bitcast_convert_type((w & 0xFFFF).astype(jnp.uint16),
                                  jnp.bfloat16)
    hi = lax.bitcast_convert_type(
        lax.shift_right_logical(w, 16).astype(jnp.uint16), jnp.bfloat16)
    return lo, hi


def _pack_pairs(ha, hb):
    ta = lax.bitcast_convert_type(
        ha.astype(jnp.bfloat16), jnp.uint16).astype(jnp.int32)
    tb = lax.bitcast_convert_type(
        hb.astype(jnp.bfloat16), jnp.uint16).astype(jnp.int32)
    return ta | lax.shift_left(tb, 16)


def _mm_compute(x_ref, a_ref, ws_ref, wna_ref, wnb_ref):
    acc = jnp.dot(x_ref[...].astype(jnp.bfloat16), ws_ref[...],
                  preferred_element_type=jnp.float32)
    lo, hi = _unpack_pairs(a_ref[...])
    acc = acc + jnp.dot(lo, wna_ref[...], preferred_element_type=jnp.float32)
    acc = acc + jnp.dot(hi, wnb_ref[...], preferred_element_type=jnp.float32)
    return jnp.maximum(acc, 0.0)


def _mm_kernel_tab(x_ref, a_ref, ws_ref, wna_ref, wnb_ref, o_ref, otab_ref):
    h = _mm_compute(x_ref, a_ref, ws_ref, wna_ref, wnb_ref)
    o_ref[...] = h
    hdim = h.shape[1]
    otab_ref[...] = _pack_pairs(h[:, :hdim // 2], h[:, hdim // 2:])


def _mm_kernel(x_ref, a_ref, ws_ref, wna_ref, wnb_ref, o_ref):
    o_ref[...] = _mm_compute(x_ref, a_ref, ws_ref, wna_ref, wnb_ref)


@functools.partial(jax.jit, static_argnames=("bm", "m_out", "emit_tab"))
def _mm(x, agg, ws, wna, wnb, bm=512, m_out=None, emit_tab=False):
    m_grid = agg.shape[0]
    d = x.shape[1]
    dw = agg.shape[1]
    h = ws.shape[1]
    m_out = m_grid if m_out is None else m_out
    out_shape = [jax.ShapeDtypeStruct((m_out, h), jnp.float32)]
    out_specs = [pl.BlockSpec((bm, h), lambda i: (i, 0))]
    if emit_tab:
        out_shape.append(jax.ShapeDtypeStruct((m_grid, h // 2), jnp.int32))
        out_specs.append(pl.BlockSpec((bm, h // 2), lambda i: (i, 0)))
    out = pl.pallas_call(
        _mm_kernel_tab if emit_tab else _mm_kernel,
        grid=(m_grid // bm,),
        in_specs=[
            pl.BlockSpec((bm, d), lambda i: (i, 0)),
            pl.BlockSpec((bm, dw), lambda i: (i, 0)),
            pl.BlockSpec((d, h), lambda i: (0, 0)),
            pl.BlockSpec((dw, h), lambda i: (0, 0)),
            pl.BlockSpec((dw, h), lambda i: (0, 0)),
        ],
        out_specs=out_specs,
        out_shape=out_shape,
    )(x, agg, ws, wna, wnb)
    return out if emit_tab else out[0]


SPLIT1 = (62, 18)
SPLIT2 = (58, 22)


def kernel(nodes, adjs, features, W1, W2):
    n, _ = adjs.shape
    d_in = features.shape[1]
    h1_dim = W1.shape[1]

    quantum = NW * NPC * 2
    n_pad = ((n + quantum - 1) // quantum) * quantum

    adjs_r = jnp.pad(adjs, ((0, n_pad - n), (0, 0))).reshape(-1, NPC * K)
    adjs_r = jnp.pad(adjs_r, ((0, 16), (0, 0)))

    dwi = d_in // 2
    tab1 = _pack_pairs(features[:, :dwi], features[:, dwi:])

    W1b = W1.astype(jnp.bfloat16)
    W2b = W2.astype(jnp.bfloat16)

    agg1 = _build_agg(n_pad, d_in, *SPLIT1)(tab1, adjs_r)
    h1, tab2 = _mm(features, agg1, W1b[:d_in], W1b[d_in:d_in + dwi],
                   W1b[d_in + dwi:], emit_tab=True)

    agg2 = _build_agg(n_pad, h1_dim, *SPLIT2)(tab2, adjs_r)
    dwh = h1_dim // 2
    return _mm(h1, agg2, W2b[:h1_dim], W2b[h1_dim:h1_dim + dwh],
               W2b[h1_dim + dwh:], m_out=n)
